# baseline (speedup 1.0000x reference)
import jax
import jax.numpy as jnp
from jax.experimental import pallas as pl
from jax.experimental.pallas import tpu as pltpu

MEM_ROWS = 65536
BATCH = 4096
FEAT_DIM = 256

F_CHUNK = 4096
NF = (MEM_ROWS - BATCH) // F_CHUNK


def _dma_body(x_ref, f_ref, o_ref, sem):
    copies = [pltpu.make_async_copy(
        x_ref, o_ref.at[pl.ds(0, BATCH)], sem.at[0])]
    for j in range(NF):
        a = BATCH + j * F_CHUNK
        copies.append(pltpu.make_async_copy(
            f_ref.at[pl.ds(a, F_CHUNK)], o_ref.at[pl.ds(a, F_CHUNK)],
            sem.at[j + 1]))
    for c in copies:
        c.start()
    for c in copies:
        c.wait()


def kernel(x, feats):
    return pl.pallas_call(
        _dma_body,
        in_specs=[
            pl.BlockSpec(memory_space=pl.ANY),
            pl.BlockSpec(memory_space=pl.ANY),
        ],
        out_specs=pl.BlockSpec(memory_space=pl.ANY),
        out_shape=jax.ShapeDtypeStruct((MEM_ROWS, FEAT_DIM), jnp.float32),
        scratch_shapes=[pltpu.SemaphoreType.DMA((NF + 1,))],
    )(x, feats)


# speedup vs baseline: 48.7971x; 48.7971x over previous
import jax
import jax.numpy as jnp
from jax.experimental import pallas as pl
from jax.experimental.pallas import tpu as pltpu

MEM_ROWS = 65536
BATCH = 4096
FEAT_DIM = 256

CHUNK = 2048
NCHUNK = MEM_ROWS // CHUNK
XCHUNK = BATCH // CHUNK
NBUF = 8
AHEAD = 4


def _dma_body(x_ref, f_ref, o_ref, buf, gsem, ssem):
    gathers, scatters = [None] * NCHUNK, [None] * NCHUNK

    def fire_gather(i):
        b = i % NBUF
        if i >= NBUF:
            scatters[i - NBUF].wait()
        src = x_ref if i < XCHUNK else f_ref
        gathers[i] = pltpu.make_async_copy(
            src.at[pl.ds(i * CHUNK, CHUNK)], buf.at[b], gsem.at[b])
        gathers[i].start()

    for i in range(AHEAD):
        fire_gather(i)
    for i in range(NCHUNK):
        if i + AHEAD < NCHUNK:
            fire_gather(i + AHEAD)
        b = i % NBUF
        gathers[i].wait()
        scatters[i] = pltpu.make_async_copy(
            buf.at[b], o_ref.at[pl.ds(i * CHUNK, CHUNK)], ssem.at[b])
        scatters[i].start()
    for i in range(NCHUNK - NBUF, NCHUNK):
        scatters[i].wait()


def kernel(x, feats):
    return pl.pallas_call(
        _dma_body,
        in_specs=[
            pl.BlockSpec(memory_space=pl.ANY),
            pl.BlockSpec(memory_space=pl.ANY),
        ],
        out_specs=pl.BlockSpec(memory_space=pl.ANY),
        out_shape=jax.ShapeDtypeStruct((MEM_ROWS, FEAT_DIM), jnp.float32),
        scratch_shapes=[
            pltpu.VMEM((NBUF, CHUNK, FEAT_DIM), jnp.float32),
            pltpu.SemaphoreType.DMA((NBUF,)),
            pltpu.SemaphoreType.DMA((NBUF,)),
        ],
    )(x, feats)
